# Initial kernel scaffold; baseline (speedup 1.0000x reference)
import jax
import jax.numpy as jnp
from jax.experimental import pallas as pl


def kernel(words_seq, table):
    raise NotImplementedError("write your pallas kernel here")


# speedup vs baseline: 6.1241x; 6.1241x over previous
import functools

import jax
import jax.numpy as jnp
from jax import lax
from jax.experimental import pallas as pl
from jax.experimental.pallas import tpu as pltpu
from jax.experimental.pallas import tpu_sc as plsc

_VVOCAB = 100000
_D = 32
_NC = 2
_NS = 16
_NW = _NC * _NS
_CHUNK = 1024


def _emb_body(table_hbm, idx_hbm, out_hbm, idx_v, rows_v, sem, *, b_per_w,
              n_chunks):
    wid = lax.axis_index("s") * _NC + lax.axis_index("c")
    base = wid * b_per_w

    def step(i, _):
        off = base + i * _CHUNK
        pltpu.sync_copy(idx_hbm.at[pl.ds(off, _CHUNK)], idx_v)
        pltpu.async_copy(table_hbm.at[idx_v], rows_v, sem).wait()
        pltpu.sync_copy(rows_v, out_hbm.at[pl.ds(off, _CHUNK)])
        return 0

    lax.fori_loop(0, n_chunks, step, 0)


@functools.lru_cache(maxsize=None)
def _make_gather(total, d):
    assert total % (_NW * _CHUNK) == 0
    b_per_w = total // _NW
    n_chunks = b_per_w // _CHUNK
    mesh = plsc.VectorSubcoreMesh(core_axis_name="c", subcore_axis_name="s")
    return pl.kernel(
        functools.partial(_emb_body, b_per_w=b_per_w, n_chunks=n_chunks),
        mesh=mesh,
        out_type=jax.ShapeDtypeStruct((total, d), jnp.float32),
        scratch_types=[
            pltpu.VMEM((_CHUNK,), jnp.int32),
            pltpu.VMEM((_CHUNK, d), jnp.float32),
            pltpu.SemaphoreType.DMA,
        ],
        compiler_params=pltpu.CompilerParams(use_tc_tiling_on_sc=False),
    )


def kernel(words_seq, table):
    b, t = words_seq.shape
    d = table.shape[1]
    idx = words_seq.reshape(-1).astype(jnp.int32)
    out = _make_gather(b * t, d)(table, idx)
    return out.reshape(b, t, d)


# speedup vs baseline: 6.4844x; 1.0588x over previous
import functools

import jax
import jax.numpy as jnp
from jax import lax
from jax.experimental import pallas as pl
from jax.experimental.pallas import tpu as pltpu
from jax.experimental.pallas import tpu_sc as plsc

_VVOCAB = 100000
_D = 32
_NC = 2
_NS = 16
_NW = _NC * _NS
_CHUNK = 1600
_NBUF = 2


def _emb_body(table_hbm, idx_hbm, out_hbm, idx_v, rows_v, sems, *, b_per_w,
              n_chunks):
    wid = lax.axis_index("s") * _NC + lax.axis_index("c")
    base = wid * b_per_w
    sem_idx, sem_gth, sem_out = sems

    def idx_start(g, b):
        pltpu.async_copy(idx_hbm.at[pl.ds(base + g * _CHUNK, _CHUNK)],
                         idx_v.at[b], sem_idx.at[b])

    for b in range(_NBUF):
        idx_start(b, b)

    def super_step(s, _):
        for b in range(_NBUF):
            g = s * _NBUF + b
            pltpu.make_async_copy(idx_hbm.at[pl.ds(0, _CHUNK)], idx_v.at[b],
                                  sem_idx.at[b]).wait()

            @pl.when(s > 0)
            def _():
                pltpu.make_async_copy(
                    rows_v.at[b], out_hbm.at[pl.ds(0, _CHUNK)],
                    sem_out.at[b]).wait()

            pltpu.async_copy(table_hbm.at[idx_v.at[b]], rows_v.at[b],
                             sem_gth.at[b]).wait()

            @pl.when(g + _NBUF < n_chunks)
            def _():
                idx_start(g + _NBUF, b)
            pltpu.async_copy(rows_v.at[b],
                             out_hbm.at[pl.ds(base + g * _CHUNK, _CHUNK)],
                             sem_out.at[b])
        return 0

    lax.fori_loop(0, n_chunks // _NBUF, super_step, 0)
    for b in range(_NBUF):
        pltpu.make_async_copy(rows_v.at[b], out_hbm.at[pl.ds(0, _CHUNK)],
                              sem_out.at[b]).wait()


@functools.lru_cache(maxsize=None)
def _make_gather(total, d):
    assert total % (_NW * _CHUNK * _NBUF) == 0
    b_per_w = total // _NW
    n_chunks = b_per_w // _CHUNK
    mesh = plsc.VectorSubcoreMesh(core_axis_name="c", subcore_axis_name="s")
    return pl.kernel(
        functools.partial(_emb_body, b_per_w=b_per_w, n_chunks=n_chunks),
        mesh=mesh,
        out_type=jax.ShapeDtypeStruct((total, d), jnp.float32),
        scratch_types=[
            pltpu.VMEM((_NBUF, _CHUNK), jnp.int32),
            pltpu.VMEM((_NBUF, _CHUNK, d), jnp.float32),
            [pltpu.SemaphoreType.DMA((_NBUF,))] * 3,
        ],
        compiler_params=pltpu.CompilerParams(use_tc_tiling_on_sc=False),
    )


def kernel(words_seq, table):
    b, t = words_seq.shape
    d = table.shape[1]
    idx = words_seq.reshape(-1).astype(jnp.int32)
    out = _make_gather(b * t, d)(table, idx)
    return out.reshape(b, t, d)
